# baseline (speedup 1.0000x reference)
import functools

import jax
import jax.numpy as jnp
from jax import lax
from jax.experimental import pallas as pl
from jax.experimental.pallas import tpu as pltpu
from jax.experimental.pallas import tpu_sc as plsc

NC = 2
NS = 16
NW = NC * NS

TB = 128
DB = 8


@functools.partial(jax.jit, static_argnames=("S", "T", "D"))
def _embedding_gather(table, idx_flat, *, S, T, D):
    n_blocks = (S * T) // TB
    blocks_per_w = n_blocks // NW
    idx_per_w = blocks_per_w * TB
    n_groups = blocks_per_w // 2
    tile_words = 8 * TB
    blk_words = DB * tile_words
    mesh = plsc.VectorSubcoreMesh(core_axis_name="c", subcore_axis_name="s")

    @functools.partial(
        pl.kernel,
        mesh=mesh,
        out_type=jax.ShapeDtypeStruct((S * T * D,), jnp.float32),
        scratch_types=[
            pltpu.VMEM((idx_per_w,), jnp.int32),
            pltpu.VMEM((TB, D), jnp.float32),
            pltpu.VMEM((TB, D), jnp.float32),
            pltpu.VMEM((blk_words,), jnp.float32),
            pltpu.VMEM((blk_words,), jnp.float32),
            pltpu.SemaphoreType.DMA,
            pltpu.SemaphoreType.DMA,
            pltpu.SemaphoreType.DMA,
            pltpu.SemaphoreType.DMA,
        ],
        compiler_params=pltpu.CompilerParams(
            use_tc_tiling_on_sc=False, needs_layout_passes=False),
    )
    def k(table_hbm, idx_hbm, out_hbm, idx_v, buf_a, buf_b, tbuf_a, tbuf_b,
          gsem_a, gsem_b, wsem_a, wsem_b):
        wid = lax.axis_index("s") * NC + lax.axis_index("c")
        pltpu.sync_copy(idx_hbm.at[pl.ds(wid * idx_per_w, idx_per_w)], idx_v)

        rows = [lax.iota(jnp.int32, 16) + (g8 * 16) for g8 in range(8)]

        def gather(k_, buf, sem):
            pltpu.async_copy(
                table_hbm.at[idx_v.at[pl.ds(k_ * TB, TB)]], buf, sem)

        def transpose(buf, tbuf):
            def db_body(db, carry):
                base = db * tile_words
                col0 = db * 8
                for dr in range(8):
                    col = jnp.full((16,), col0 + dr, jnp.int32)
                    for g8 in range(8):
                        v = plsc.load_gather(buf, [rows[g8], col])
                        tbuf[pl.ds(base + dr * TB + g8 * 16, 16)] = v
                return carry
            lax.fori_loop(0, DB, db_body, 0)

        def writeback(k_, tbuf, sem):
            blk = wid * blocks_per_w + k_
            p = blk // TB
            bb = blk % TB
            base = p * (D * S) + bb * tile_words
            for db in range(DB):
                pltpu.async_copy(
                    tbuf.at[pl.ds(db * tile_words, tile_words)],
                    out_hbm.at[pl.ds(base + db * (tile_words * TB),
                                     tile_words)],
                    sem)

        def wait_fill(buf, sem):
            pltpu.make_async_copy(table_hbm.at[pl.ds(0, TB)], buf, sem).wait()

        def wait_drain(tbuf, sem):
            for _ in range(DB):
                pltpu.make_async_copy(
                    tbuf.at[pl.ds(0, tile_words)],
                    out_hbm.at[pl.ds(0, tile_words)], sem).wait()

        gather(0, buf_a, gsem_a)
        gather(1, buf_b, gsem_b)

        def body(g, carry):
            for b, (buf, tbuf, gsem, wsem) in enumerate(
                    ((buf_a, tbuf_a, gsem_a, wsem_a),
                     (buf_b, tbuf_b, gsem_b, wsem_b))):
                k_ = 2 * g + b
                wait_fill(buf, gsem)

                @pl.when(g > 0)
                def _():
                    wait_drain(tbuf, wsem)

                transpose(buf, tbuf)

                @pl.when(g < n_groups - 1)
                def _():
                    gather(k_ + 2, buf, gsem)

                writeback(k_, tbuf, wsem)
            return carry

        lax.fori_loop(0, n_groups, body, 0)
        wait_drain(tbuf_a, wsem_a)
        wait_drain(tbuf_b, wsem_b)

    return k(table, idx_flat)


def kernel(token_ids, weight):
    S, T = token_ids.shape
    D = weight.shape[1]
    idx_flat = token_ids.T.reshape(S * T).astype(jnp.int32)
    out1d = _embedding_gather(weight, idx_flat, S=S, T=T, D=D)
    x5 = out1d.reshape(T, D // 8, S // 128, 8, 128)
    return x5.transpose(2, 4, 0, 1, 3).reshape(S, T, D)


# speedup vs baseline: 1.8483x; 1.8483x over previous
import functools

import jax
import jax.numpy as jnp
from jax import lax
from jax.experimental import pallas as pl
from jax.experimental.pallas import tpu as pltpu
from jax.experimental.pallas import tpu_sc as plsc

NC = 2
NS = 16
NW = NC * NS

TB = 128
DB = 8
SKEW = 129


@functools.partial(jax.jit, static_argnames=("S", "T", "D"))
def _embedding_gather(table, idx_flat, *, S, T, D):
    n_blocks = (S * T) // TB
    blocks_per_w = n_blocks // NW
    idx_per_w = blocks_per_w * TB
    n_groups = blocks_per_w // 2
    mesh = plsc.VectorSubcoreMesh(core_axis_name="c", subcore_axis_name="s")

    @functools.partial(
        pl.kernel,
        mesh=mesh,
        out_type=jax.ShapeDtypeStruct((T, DB, S // TB, 8, TB), jnp.float32),
        scratch_types=[
            pltpu.VMEM((idx_per_w,), jnp.int32),
            pltpu.VMEM((TB, D), jnp.float32),
            pltpu.VMEM((TB, D), jnp.float32),
            pltpu.VMEM((D, SKEW), jnp.float32),
            pltpu.VMEM((D, SKEW), jnp.float32),
            pltpu.SemaphoreType.DMA,
            pltpu.SemaphoreType.DMA,
            pltpu.SemaphoreType.DMA,
            pltpu.SemaphoreType.DMA,
        ],
        compiler_params=pltpu.CompilerParams(
            use_tc_tiling_on_sc=False, needs_layout_passes=False),
    )
    def k(table_hbm, idx_hbm, out_hbm, idx_v, buf_a, buf_b, sbuf_a, sbuf_b,
          gsem_a, gsem_b, wsem_a, wsem_b):
        wid = lax.axis_index("s") * NC + lax.axis_index("c")
        pltpu.sync_copy(idx_hbm.at[pl.ds(wid * idx_per_w, idx_per_w)], idx_v)

        iota = lax.iota(jnp.int32, 16)
        row_idx = [iota + 16 * c for c in range(D // 16)]

        def gather(k_, buf, sem):
            pltpu.async_copy(
                table_hbm.at[idx_v.at[pl.ds(k_ * TB, TB)]], buf, sem)

        def transpose(buf, sbuf):
            def bl_body(bl0, carry):
                for blo in range(4):
                    bl = bl0 * 4 + blo
                    col = jnp.full((16,), bl, jnp.int32)
                    for c in range(D // 16):
                        v = buf[bl, pl.ds(c * 16, 16)]
                        plsc.store_scatter(sbuf, [row_idx[c], col], v)
                return carry
            lax.fori_loop(0, TB // 4, bl_body, 0)

        def writeback(k_, sbuf, sem):
            blk = wid * blocks_per_w + k_
            p = blk // TB
            bb = blk % TB
            for db in range(DB):
                pltpu.async_copy(
                    sbuf.at[pl.ds(db * 8, 8), pl.ds(0, TB)],
                    out_hbm.at[p, db, bb], sem)

        def wait_fill(buf, sem):
            pltpu.make_async_copy(table_hbm.at[pl.ds(0, TB)], buf, sem).wait()

        def wait_drain(sbuf, sem):
            for _ in range(DB):
                pltpu.make_async_copy(
                    sbuf.at[pl.ds(0, 8), pl.ds(0, TB)],
                    out_hbm.at[0, 0, 0], sem).wait()

        gather(0, buf_a, gsem_a)
        gather(1, buf_b, gsem_b)

        def body(g, carry):
            for b, (buf, sbuf, gsem, wsem) in enumerate(
                    ((buf_a, sbuf_a, gsem_a, wsem_a),
                     (buf_b, sbuf_b, gsem_b, wsem_b))):
                k_ = 2 * g + b
                wait_fill(buf, gsem)

                @pl.when(g > 0)
                def _():
                    wait_drain(sbuf, wsem)

                transpose(buf, sbuf)

                @pl.when(g < n_groups - 1)
                def _():
                    gather(k_ + 2, buf, gsem)

                writeback(k_, sbuf, wsem)
            return carry

        lax.fori_loop(0, n_groups, body, 0)
        wait_drain(sbuf_a, wsem_a)
        wait_drain(sbuf_b, wsem_b)

    return k(table, idx_flat)


def kernel(token_ids, weight):
    S, T = token_ids.shape
    D = weight.shape[1]
    idx_flat = token_ids.T.reshape(S * T).astype(jnp.int32)
    x5 = _embedding_gather(weight, idx_flat, S=S, T=T, D=D)
    return x5.transpose(2, 4, 0, 1, 3).reshape(S, T, D)


# speedup vs baseline: 1.8580x; 1.0052x over previous
import functools

import jax
import jax.numpy as jnp
from jax import lax
from jax.experimental import pallas as pl
from jax.experimental.pallas import tpu as pltpu
from jax.experimental.pallas import tpu_sc as plsc

NC = 2
NS = 16
NW = NC * NS

TB = 128
DB = 8
SKEW = 129


@functools.partial(jax.jit, static_argnames=("S", "T", "D"))
def _embedding_gather(table, idx_flat, *, S, T, D):
    n_blocks = (S * T) // TB
    blocks_per_w = n_blocks // NW
    idx_per_w = blocks_per_w * TB
    n_groups = blocks_per_w // 2
    mesh = plsc.VectorSubcoreMesh(core_axis_name="c", subcore_axis_name="s")

    @functools.partial(
        pl.kernel,
        mesh=mesh,
        out_type=jax.ShapeDtypeStruct((T, DB, S // TB, 8, TB), jnp.float32),
        scratch_types=[
            pltpu.VMEM((idx_per_w,), jnp.int32),
            pltpu.VMEM((TB, D), jnp.float32),
            pltpu.VMEM((TB, D), jnp.float32),
            pltpu.VMEM((D, SKEW), jnp.float32),
            pltpu.VMEM((D, SKEW), jnp.float32),
            pltpu.SemaphoreType.DMA,
            pltpu.SemaphoreType.DMA,
            pltpu.SemaphoreType.DMA,
            pltpu.SemaphoreType.DMA,
        ],
        compiler_params=pltpu.CompilerParams(
            use_tc_tiling_on_sc=False, needs_layout_passes=False),
    )
    def k(table_hbm, idx_hbm, out_hbm, idx_v, buf_a, buf_b, sbuf_a, sbuf_b,
          gsem_a, gsem_b, wsem_a, wsem_b):
        wid = lax.axis_index("s") * NC + lax.axis_index("c")
        pltpu.sync_copy(idx_hbm.at[pl.ds(wid * idx_per_w, idx_per_w)], idx_v)

        iota = lax.iota(jnp.int32, 16)
        row_idx = [iota + 16 * c for c in range(D // 16)]

        def gather(k_, buf, sem):
            pltpu.async_copy(
                table_hbm.at[idx_v.at[pl.ds(k_ * TB, TB)]], buf, sem)

        def transpose(buf, sbuf):
            def bl_body(bl0, carry):
                for blo in range(8):
                    bl = bl0 * 8 + blo
                    col = jnp.full((16,), bl, jnp.int32)
                    for c in range(D // 16):
                        v = buf[bl, pl.ds(c * 16, 16)]
                        plsc.store_scatter(sbuf, [row_idx[c], col], v)
                return carry
            lax.fori_loop(0, TB // 8, bl_body, 0)

        def writeback(k_, sbuf, sem):
            blk = wid * blocks_per_w + k_
            p = blk // TB
            bb = blk % TB
            for db in range(DB):
                pltpu.async_copy(
                    sbuf.at[pl.ds(db * 8, 8), pl.ds(0, TB)],
                    out_hbm.at[p, db, bb], sem)

        def wait_fill(buf, sem):
            pltpu.make_async_copy(table_hbm.at[pl.ds(0, TB)], buf, sem).wait()

        def wait_drain(sbuf, sem):
            for _ in range(DB):
                pltpu.make_async_copy(
                    sbuf.at[pl.ds(0, 8), pl.ds(0, TB)],
                    out_hbm.at[0, 0, 0], sem).wait()

        gather(0, buf_a, gsem_a)
        gather(1, buf_b, gsem_b)

        def body(g, carry):
            for b, (buf, sbuf, gsem, wsem) in enumerate(
                    ((buf_a, sbuf_a, gsem_a, wsem_a),
                     (buf_b, sbuf_b, gsem_b, wsem_b))):
                k_ = 2 * g + b
                wait_fill(buf, gsem)

                @pl.when(g > 0)
                def _():
                    wait_drain(sbuf, wsem)

                transpose(buf, sbuf)

                @pl.when(g < n_groups - 1)
                def _():
                    gather(k_ + 2, buf, gsem)

                writeback(k_, sbuf, wsem)
            return carry

        lax.fori_loop(0, n_groups, body, 0)
        wait_drain(sbuf_a, wsem_a)
        wait_drain(sbuf_b, wsem_b)

    return k(table, idx_flat)


def kernel(token_ids, weight):
    S, T = token_ids.shape
    D = weight.shape[1]
    idx_flat = token_ids.T.reshape(S * T).astype(jnp.int32)
    x5 = _embedding_gather(weight, idx_flat, S=S, T=T, D=D)
    return x5.transpose(2, 4, 0, 1, 3).reshape(S, T, D)
